# baseline (speedup 1.0000x reference)
import functools

import jax
import jax.numpy as jnp
from jax import lax
from jax.experimental import pallas as pl
from jax.experimental.pallas import tpu as pltpu
from jax.experimental.pallas import tpu_sc as plsc

GRID = 64
DIM = 64
LANES = 16
NC = 2
NS = 16
NW = NC * NS
DCH = DIM // LANES
NPG = NW // DCH
CHUNK = 512
NSUB = CHUNK // LANES
NBUF = 2


def _make_interp(n_points: int):
    ppg = n_points // NPG
    iters = ppg // CHUNK
    mesh = plsc.VectorSubcoreMesh(core_axis_name="c", subcore_axis_name="s")

    @functools.partial(
        pl.kernel,
        mesh=mesh,
        out_type=jax.ShapeDtypeStruct((n_points, DIM), jnp.float32),
        scratch_types=[
            [pltpu.VMEM((2 * CHUNK,), jnp.float32) for _ in range(NBUF)],
            [pltpu.VMEM((4 * CHUNK,), jnp.int32) for _ in range(NBUF)],
            [pltpu.VMEM((4 * CHUNK,), jnp.float32) for _ in range(NBUF)],
            [pltpu.VMEM((4 * CHUNK, LANES), jnp.float32) for _ in range(NBUF)],
            [pltpu.VMEM((CHUNK, LANES), jnp.float32) for _ in range(NBUF)],
            [pltpu.SemaphoreType.DMA for _ in range(NBUF)],
            [pltpu.SemaphoreType.DMA for _ in range(NBUF)],
            [pltpu.SemaphoreType.DMA for _ in range(NBUF)],
        ],
        compiler_params=pltpu.CompilerParams(
            use_tc_tiling_on_sc=False, needs_layout_passes=False
        ),
    )
    def interp(
        coords_hbm, table_hbm, out_hbm,
        crd_v, idx_v, w_v, rows_v, out_v, sin, sg, sout,
    ):
        wid = lax.axis_index("s") * NC + lax.axis_index("c")
        g = wid % DCH
        pg = wid // DCH
        p_base = pg * ppg
        lanes = lax.iota(jnp.int32, LANES)

        def in_copy(it, b):
            p0 = p_base + it * CHUNK
            return pltpu.make_async_copy(
                coords_hbm.at[pl.ds(p0 * 2, 2 * CHUNK)], crd_v[b], sin[b]
            )

        def row_gather(b):
            return pltpu.make_async_copy(
                table_hbm.at[idx_v[b]], rows_v[b], sg[b]
            )

        def out_copy(it, b):
            p0 = p_base + it * CHUNK
            return pltpu.make_async_copy(
                out_v[b],
                out_hbm.at[pl.ds(p0, CHUNK), pl.ds(g * LANES, LANES)],
                sout[b],
            )

        def build(b):
            base_row = g * (GRID * GRID)

            @plsc.parallel_loop(0, NSUB, 1)
            def _build(s):
                idx2 = lanes * 2 + s * (2 * LANES)
                xs = plsc.load_gather(crd_v[b], [idx2])
                ys = plsc.load_gather(crd_v[b], [idx2 + 1])
                cx = xs * jnp.float32(GRID - 1)
                cy = ys * jnp.float32(GRID - 1)
                xi = jnp.minimum(jnp.maximum(cx.astype(jnp.int32), 0), GRID - 2)
                yi = jnp.minimum(jnp.maximum(cy.astype(jnp.int32), 0), GRID - 2)
                fx = cx - xi.astype(jnp.float32)
                fy = cy - yi.astype(jnp.float32)
                gx = 1.0 - fx
                gy = 1.0 - fy
                r00 = xi * GRID + yi + base_row
                o = s * (4 * LANES)
                idx_v[b][pl.ds(o, LANES)] = r00
                idx_v[b][pl.ds(o + LANES, LANES)] = r00 + 1
                idx_v[b][pl.ds(o + 2 * LANES, LANES)] = r00 + GRID
                idx_v[b][pl.ds(o + 3 * LANES, LANES)] = r00 + (GRID + 1)
                w_v[b][pl.ds(o, LANES)] = gx * gy
                w_v[b][pl.ds(o + LANES, LANES)] = gx * fy
                w_v[b][pl.ds(o + 2 * LANES, LANES)] = fx * gy
                w_v[b][pl.ds(o + 3 * LANES, LANES)] = fx * fy

        bcast_dnums = lax.GatherDimensionNumbers(
            offset_dims=(), collapsed_slice_dims=(0,), start_index_map=(0,)
        )

        def bcast(v, p):
            idx = jnp.full((LANES, 1), p, dtype=jnp.int32)
            return lax.gather(
                v, idx, bcast_dnums, (1,),
                mode=lax.GatherScatterMode.PROMISE_IN_BOUNDS,
            )

        def combine(b):
            @plsc.parallel_loop(0, NSUB, 1)
            def _combine(s):
                o = s * (4 * LANES)
                w00v = w_v[b][pl.ds(o, LANES)]
                w01v = w_v[b][pl.ds(o + LANES, LANES)]
                w10v = w_v[b][pl.ds(o + 2 * LANES, LANES)]
                w11v = w_v[b][pl.ds(o + 3 * LANES, LANES)]
                for p in range(LANES):
                    w00 = bcast(w00v, p)
                    w01 = bcast(w01v, p)
                    w10 = bcast(w10v, p)
                    w11 = bcast(w11v, p)
                    v00 = rows_v[b][o + p, :]
                    v01 = rows_v[b][o + LANES + p, :]
                    v10 = rows_v[b][o + 2 * LANES + p, :]
                    v11 = rows_v[b][o + 3 * LANES + p, :]
                    r = v00 * w00 + v01 * w01 + v10 * w10 + v11 * w11
                    out_v[b][s * LANES + p, :] = r

        in_copy(0, 0).start()
        in_copy(1, 1).start()
        in_copy(0, 0).wait()
        build(0)
        row_gather(0).start()

        def pair_body(it2, carry):
            for par in range(2):
                it = it2 * 2 + par + 1
                bc = (par + 1) % 2
                bp = par % 2

                @pl.when(it < iters)
                def _steady():
                    in_copy(it, bc).wait()

                    @pl.when(it + 1 < iters)
                    def _pref():
                        in_copy(it + 1, bp).start()

                    row_gather(bp).wait()

                    @pl.when(it >= 3)
                    def _drain():
                        out_copy(it - 3, bp).wait()

                    combine(bp)
                    out_copy(it - 1, bp).start()
                    build(bc)
                    row_gather(bc).start()

                @pl.when(it == iters)
                def _tail():
                    row_gather(bp).wait()

                    @pl.when(it >= 3)
                    def _drain2():
                        out_copy(it - 3, bp).wait()

                    combine(bp)
                    out_copy(it - 1, bp).start()

            return carry

        lax.fori_loop(0, (iters + 1) // 2, pair_body, 0, unroll=False)
        out_copy(iters - 2, (iters - 2) % 2).wait()
        out_copy(iters - 1, (iters - 1) % 2).wait()

    return interp


def kernel(coords, embedding):
    b, l, _ = coords.shape
    n = b * l
    assert embedding.shape == (GRID, GRID, DIM)
    assert n % (NPG * CHUNK) == 0
    cflat = coords.reshape(n * 2)
    table = (
        embedding.reshape(GRID * GRID, DCH, LANES)
        .transpose(1, 0, 2)
        .reshape(DCH * GRID * GRID, LANES)
    )
    out = _make_interp(n)(cflat, table)
    return out.reshape(b, l, DIM)


# speedup vs baseline: 1.0926x; 1.0926x over previous
import functools

import jax
import jax.numpy as jnp
from jax import lax
from jax.experimental import pallas as pl
from jax.experimental.pallas import tpu as pltpu
from jax.experimental.pallas import tpu_sc as plsc

GRID = 64
DIM = 64
LANES = 16
NC = 2
NS = 16
NW = NC * NS
DCH = DIM // LANES
NPG = NW // DCH
CHUNK = 1024
NBUF = 2


def _make_interp(n_points: int):
    ppg = n_points // NPG
    iters = ppg // CHUNK
    assert iters % NBUF == 0
    mesh = plsc.VectorSubcoreMesh(core_axis_name="c", subcore_axis_name="s")

    @functools.partial(
        pl.kernel,
        mesh=mesh,
        out_type=jax.ShapeDtypeStruct((n_points, DIM), jnp.float32),
        scratch_types=[
            pltpu.VMEM((GRID * GRID * LANES,), jnp.float32),
            [pltpu.VMEM((2 * CHUNK,), jnp.float32) for _ in range(NBUF)],
            [pltpu.VMEM((CHUNK, LANES), jnp.float32) for _ in range(NBUF)],
            [pltpu.SemaphoreType.DMA for _ in range(NBUF)],
            [pltpu.SemaphoreType.DMA for _ in range(NBUF)],
        ],
        compiler_params=pltpu.CompilerParams(
            use_tc_tiling_on_sc=False, needs_layout_passes=False
        ),
    )
    def interp(coords_hbm, table_hbm, out_hbm, tab_v, crd_v, out_v, sin, sout):
        wid = lax.axis_index("s") * NC + lax.axis_index("c")
        g = wid % DCH
        pg = wid // DCH
        p_base = pg * ppg

        pltpu.sync_copy(table_hbm.at[g], tab_v)

        lanes = lax.iota(jnp.int32, LANES)

        def in_copy(it, b):
            p0 = p_base + it * CHUNK
            return pltpu.make_async_copy(
                coords_hbm.at[pl.ds(p0 * 2, 2 * CHUNK)], crd_v[b], sin[b]
            )

        def out_copy(it, b):
            p0 = p_base + it * CHUNK
            return pltpu.make_async_copy(
                out_v[b],
                out_hbm.at[pl.ds(p0, CHUNK), pl.ds(g * LANES, LANES)],
                sout[b],
            )

        def compute(it, b):
            @plsc.parallel_loop(0, CHUNK // LANES, 1)
            def sub_body(s):
                idx2 = lanes * 2 + s * (2 * LANES)
                xs = plsc.load_gather(crd_v[b], [idx2])
                ys = plsc.load_gather(crd_v[b], [idx2 + 1])
                cx = xs * jnp.float32(GRID - 1)
                cy = ys * jnp.float32(GRID - 1)
                xi = jnp.minimum(jnp.maximum(cx.astype(jnp.int32), 0), GRID - 2)
                yi = jnp.minimum(jnp.maximum(cy.astype(jnp.int32), 0), GRID - 2)
                fx = cx - xi.astype(jnp.float32)
                fy = cy - yi.astype(jnp.float32)
                gx = 1.0 - fx
                gy = 1.0 - fy
                w00 = gx * gy
                w01 = gx * fy
                w10 = fx * gy
                w11 = fx * fy
                e00 = (xi * GRID + yi) * LANES
                row = lanes + s * LANES
                for j in range(LANES):
                    rot = (lanes + j) & (LANES - 1)
                    v00 = plsc.load_gather(tab_v, [e00 + rot])
                    v01 = plsc.load_gather(tab_v, [e00 + (rot + LANES)])
                    v10 = plsc.load_gather(tab_v, [e00 + (rot + GRID * LANES)])
                    v11 = plsc.load_gather(
                        tab_v, [e00 + (rot + (GRID * LANES + LANES))]
                    )
                    r = v00 * w00 + v01 * w01 + v10 * w10 + v11 * w11
                    plsc.store_scatter(out_v[b], [row, rot], r)

        for b in range(NBUF):
            in_copy(b, b).start()

        def stage(it2, carry):
            for b in range(NBUF):
                it = it2 * NBUF + b
                in_copy(it, b).wait()

                @pl.when(it2 >= 1)
                def _drain():
                    out_copy(it - NBUF, b).wait()

                compute(it, b)
                out_copy(it, b).start()

                @pl.when(it + NBUF < iters)
                def _prefetch():
                    in_copy(it + NBUF, b).start()

            return carry

        lax.fori_loop(0, iters // NBUF, stage, 0, unroll=False)
        for b in range(NBUF):
            out_copy(iters - NBUF + b, b).wait()

    return interp


def kernel(coords, embedding):
    b, l, _ = coords.shape
    n = b * l
    assert embedding.shape == (GRID, GRID, DIM)
    assert n % (NPG * CHUNK) == 0
    cflat = coords.reshape(n * 2)
    table = (
        embedding.reshape(GRID * GRID, DCH, LANES)
        .transpose(1, 0, 2)
        .reshape(DCH, GRID * GRID * LANES)
    )
    out = _make_interp(n)(cflat, table)
    return out.reshape(b, l, DIM)


# speedup vs baseline: 1.1098x; 1.0158x over previous
import functools

import jax
import jax.numpy as jnp
from jax import lax
from jax.experimental import pallas as pl
from jax.experimental.pallas import tpu as pltpu
from jax.experimental.pallas import tpu_sc as plsc

GRID = 64
DIM = 64
LANES = 16
NC = 2
NS = 16
NW = NC * NS
DCH = DIM // LANES
NPG = NW // DCH
CHUNK = 512
NBUF = 4


def _make_interp(n_points: int):
    ppg = n_points // NPG
    iters = ppg // CHUNK
    assert iters % NBUF == 0
    mesh = plsc.VectorSubcoreMesh(core_axis_name="c", subcore_axis_name="s")

    @functools.partial(
        pl.kernel,
        mesh=mesh,
        out_type=jax.ShapeDtypeStruct((n_points, DIM), jnp.float32),
        scratch_types=[
            pltpu.VMEM((GRID * GRID * LANES,), jnp.float32),
            [pltpu.VMEM((2 * CHUNK,), jnp.float32) for _ in range(NBUF)],
            [pltpu.VMEM((CHUNK, LANES), jnp.float32) for _ in range(NBUF)],
            [pltpu.SemaphoreType.DMA for _ in range(NBUF)],
            [pltpu.SemaphoreType.DMA for _ in range(NBUF)],
        ],
        compiler_params=pltpu.CompilerParams(
            use_tc_tiling_on_sc=False, needs_layout_passes=False
        ),
    )
    def interp(coords_hbm, table_hbm, out_hbm, tab_v, crd_v, out_v, sin, sout):
        wid = lax.axis_index("s") * NC + lax.axis_index("c")
        g = wid % DCH
        pg = wid // DCH
        p_base = pg * ppg

        pltpu.sync_copy(table_hbm.at[g], tab_v)

        lanes = lax.iota(jnp.int32, LANES)

        def in_copy(it, b):
            p0 = p_base + it * CHUNK
            return pltpu.make_async_copy(
                coords_hbm.at[pl.ds(p0 * 2, 2 * CHUNK)], crd_v[b], sin[b]
            )

        def out_copy(it, b):
            p0 = p_base + it * CHUNK
            return pltpu.make_async_copy(
                out_v[b],
                out_hbm.at[pl.ds(p0, CHUNK), pl.ds(g * LANES, LANES)],
                sout[b],
            )

        def compute(it, b):
            @plsc.parallel_loop(0, CHUNK // LANES, 1)
            def sub_body(s):
                idx2 = lanes * 2 + s * (2 * LANES)
                xs = plsc.load_gather(crd_v[b], [idx2])
                ys = plsc.load_gather(crd_v[b], [idx2 + 1])
                cx = xs * jnp.float32(GRID - 1)
                cy = ys * jnp.float32(GRID - 1)
                xi = jnp.minimum(jnp.maximum(cx.astype(jnp.int32), 0), GRID - 2)
                yi = jnp.minimum(jnp.maximum(cy.astype(jnp.int32), 0), GRID - 2)
                fx = cx - xi.astype(jnp.float32)
                fy = cy - yi.astype(jnp.float32)
                gx = 1.0 - fx
                gy = 1.0 - fy
                w00 = gx * gy
                w01 = gx * fy
                w10 = fx * gy
                w11 = fx * fy
                e00 = (xi * GRID + yi) * LANES
                row = lanes + s * LANES
                for j in range(LANES):
                    rot = (lanes + j) & (LANES - 1)
                    v00 = plsc.load_gather(tab_v, [e00 + rot])
                    v01 = plsc.load_gather(tab_v, [e00 + (rot + LANES)])
                    v10 = plsc.load_gather(tab_v, [e00 + (rot + GRID * LANES)])
                    v11 = plsc.load_gather(
                        tab_v, [e00 + (rot + (GRID * LANES + LANES))]
                    )
                    r = v00 * w00 + v01 * w01 + v10 * w10 + v11 * w11
                    plsc.store_scatter(out_v[b], [row, rot], r)

        for b in range(NBUF):
            in_copy(b, b).start()

        def stage(it2, carry):
            for b in range(NBUF):
                it = it2 * NBUF + b
                in_copy(it, b).wait()

                @pl.when(it2 >= 1)
                def _drain():
                    out_copy(it - NBUF, b).wait()

                compute(it, b)
                out_copy(it, b).start()

                @pl.when(it + NBUF < iters)
                def _prefetch():
                    in_copy(it + NBUF, b).start()

            return carry

        lax.fori_loop(0, iters // NBUF, stage, 0, unroll=False)
        for b in range(NBUF):
            out_copy(iters - NBUF + b, b).wait()

    return interp


def kernel(coords, embedding):
    b, l, _ = coords.shape
    n = b * l
    assert embedding.shape == (GRID, GRID, DIM)
    assert n % (NPG * CHUNK) == 0
    cflat = coords.reshape(n * 2)
    table = (
        embedding.reshape(GRID * GRID, DCH, LANES)
        .transpose(1, 0, 2)
        .reshape(DCH, GRID * GRID * LANES)
    )
    out = _make_interp(n)(cflat, table)
    return out.reshape(b, l, DIM)


# speedup vs baseline: 1.1137x; 1.0035x over previous
import functools

import jax
import jax.numpy as jnp
from jax import lax
from jax.experimental import pallas as pl
from jax.experimental.pallas import tpu as pltpu
from jax.experimental.pallas import tpu_sc as plsc

GRID = 64
DIM = 64
LANES = 16
NC = 2
NS = 16
NW = NC * NS
DCH = DIM // LANES
NPG = NW // DCH
CHUNK = 640
NBUF = 4


def _make_interp(n_points: int):
    ppg = n_points // NPG
    iters = ppg // CHUNK
    assert iters % NBUF == 0
    mesh = plsc.VectorSubcoreMesh(core_axis_name="c", subcore_axis_name="s")

    @functools.partial(
        pl.kernel,
        mesh=mesh,
        out_type=jax.ShapeDtypeStruct((n_points, DIM), jnp.float32),
        scratch_types=[
            pltpu.VMEM((GRID * GRID * LANES,), jnp.float32),
            [pltpu.VMEM((2 * CHUNK,), jnp.float32) for _ in range(NBUF)],
            [pltpu.VMEM((CHUNK, LANES), jnp.float32) for _ in range(NBUF)],
            [pltpu.SemaphoreType.DMA for _ in range(NBUF)],
            [pltpu.SemaphoreType.DMA for _ in range(NBUF)],
        ],
        compiler_params=pltpu.CompilerParams(
            use_tc_tiling_on_sc=False, needs_layout_passes=False
        ),
    )
    def interp(coords_hbm, table_hbm, out_hbm, tab_v, crd_v, out_v, sin, sout):
        wid = lax.axis_index("s") * NC + lax.axis_index("c")
        g = wid % DCH
        pg = wid // DCH
        p_base = pg * ppg

        pltpu.sync_copy(table_hbm.at[g], tab_v)

        lanes = lax.iota(jnp.int32, LANES)

        def in_copy(it, b):
            p0 = p_base + it * CHUNK
            return pltpu.make_async_copy(
                coords_hbm.at[pl.ds(p0 * 2, 2 * CHUNK)], crd_v[b], sin[b]
            )

        def out_copy(it, b):
            p0 = p_base + it * CHUNK
            return pltpu.make_async_copy(
                out_v[b],
                out_hbm.at[pl.ds(p0, CHUNK), pl.ds(g * LANES, LANES)],
                sout[b],
            )

        def compute(it, b):
            @plsc.parallel_loop(0, CHUNK // LANES, 1)
            def sub_body(s):
                idx2 = lanes * 2 + s * (2 * LANES)
                xs = plsc.load_gather(crd_v[b], [idx2])
                ys = plsc.load_gather(crd_v[b], [idx2 + 1])
                cx = xs * jnp.float32(GRID - 1)
                cy = ys * jnp.float32(GRID - 1)
                xi = jnp.minimum(jnp.maximum(cx.astype(jnp.int32), 0), GRID - 2)
                yi = jnp.minimum(jnp.maximum(cy.astype(jnp.int32), 0), GRID - 2)
                fx = cx - xi.astype(jnp.float32)
                fy = cy - yi.astype(jnp.float32)
                gx = 1.0 - fx
                gy = 1.0 - fy
                w00 = gx * gy
                w01 = gx * fy
                w10 = fx * gy
                w11 = fx * fy
                e00 = (xi * GRID + yi) * LANES
                row = lanes + s * LANES
                for j in range(LANES):
                    rot = (lanes + j) & (LANES - 1)
                    v00 = plsc.load_gather(tab_v, [e00 + rot])
                    v01 = plsc.load_gather(tab_v, [e00 + (rot + LANES)])
                    v10 = plsc.load_gather(tab_v, [e00 + (rot + GRID * LANES)])
                    v11 = plsc.load_gather(
                        tab_v, [e00 + (rot + (GRID * LANES + LANES))]
                    )
                    r = v00 * w00 + v01 * w01 + v10 * w10 + v11 * w11
                    plsc.store_scatter(out_v[b], [row, rot], r)

        for b in range(NBUF):
            in_copy(b, b).start()

        def stage(it2, carry):
            for b in range(NBUF):
                it = it2 * NBUF + b
                in_copy(it, b).wait()

                @pl.when(it2 >= 1)
                def _drain():
                    out_copy(it - NBUF, b).wait()

                compute(it, b)
                out_copy(it, b).start()

                @pl.when(it + NBUF < iters)
                def _prefetch():
                    in_copy(it + NBUF, b).start()

            return carry

        lax.fori_loop(0, iters // NBUF, stage, 0, unroll=False)
        for b in range(NBUF):
            out_copy(iters - NBUF + b, b).wait()

    return interp


def kernel(coords, embedding):
    b, l, _ = coords.shape
    n = b * l
    assert embedding.shape == (GRID, GRID, DIM)
    assert n % (NPG * CHUNK) == 0
    cflat = coords.reshape(n * 2)
    table = (
        embedding.reshape(GRID * GRID, DCH, LANES)
        .transpose(1, 0, 2)
        .reshape(DCH, GRID * GRID * LANES)
    )
    out = _make_interp(n)(cflat, table)
    return out.reshape(b, l, DIM)
